# Initial kernel scaffold; baseline (speedup 1.0000x reference)
import jax
import jax.numpy as jnp
from jax.experimental import pallas as pl


def kernel(memory_matrix, usage_vector, read_weights, write_weight_prev, free_gates, lookup_keys, strengths, write_gate, allocation_gate, write_vector, erase_vector):
    raise NotImplementedError("write your pallas kernel here")


# speedup vs baseline: 1.2617x; 1.2617x over previous
import functools

import jax
import jax.numpy as jnp
from jax import lax
from jax.experimental import pallas as pl
from jax.experimental.pallas import tpu as pltpu
from jax.experimental.pallas import tpu_sc as plsc

B, S, D, H = 32, 16384, 64, 4
LANES = 16
BLK = S // LANES
BS = 4096
NSB = S // BS
EPS = 1e-8
STICKY_THR = 1e-35


def _k1_body(mem_ref, key_ref, str_ref, u_ref, w_ref, rw_ref, fg_ref,
             uu_ref, z_ref, m_ref, l_ref):
    ns = pl.program_id(1)
    mem = mem_ref[0]
    key = key_ref[0, 0]
    kn = key / (jnp.sqrt(jnp.sum(key * key)) + EPS)
    nr = jnp.sqrt(jnp.sum(mem * mem, axis=1, keepdims=True))
    mem_n = mem / (nr + EPS)
    cos = jnp.dot(mem_n, kn[:, None],
                  preferred_element_type=jnp.float32)[:, 0]
    z = cos * str_ref[0, 0, 0]
    z_ref[0, 0] = z

    rw = rw_ref[0]
    fg = fg_ref[0, 0]
    t = 2.0 - rw * fg[None, :]
    ret = t[:, 0] * t[:, 1] * t[:, 2] * t[:, 3]
    u = u_ref[0, 0]
    w = w_ref[0, 0]
    uu_ref[0, 0] = (u + w - u * w) * ret

    bm = jnp.max(z)
    bl = jnp.sum(jnp.exp(z - bm))

    @pl.when(ns == 0)
    def _():
        m_ref[0, 0, 0] = bm
        l_ref[0, 0, 0] = bl

    @pl.when(ns != 0)
    def _():
        m_prev = m_ref[0, 0, 0]
        l_prev = l_ref[0, 0, 0]
        m_new = jnp.maximum(m_prev, bm)
        l_ref[0, 0, 0] = (l_prev * jnp.exp(m_prev - m_new)
                          + bl * jnp.exp(bm - m_new))
        m_ref[0, 0, 0] = m_new


def _k1_call(memory_matrix, lookup_keys3d, strengths3d, usage3d,
             write_prev3d, read_weights, free_gates3d):
    return pl.pallas_call(
        _k1_body,
        grid=(B, NSB),
        in_specs=[
            pl.BlockSpec((1, BS, D), lambda b, n: (b, n, 0)),
            pl.BlockSpec((1, 1, D), lambda b, n: (b, 0, 0)),
            pl.BlockSpec((1, 1, 1), lambda b, n: (b, 0, 0), memory_space=pltpu.SMEM),
            pl.BlockSpec((1, 1, BS), lambda b, n: (b, 0, n)),
            pl.BlockSpec((1, 1, BS), lambda b, n: (b, 0, n)),
            pl.BlockSpec((1, BS, H), lambda b, n: (b, n, 0)),
            pl.BlockSpec((1, 1, H), lambda b, n: (b, 0, 0)),
        ],
        out_specs=[
            pl.BlockSpec((1, 1, BS), lambda b, n: (b, 0, n)),
            pl.BlockSpec((1, 1, BS), lambda b, n: (b, 0, n)),
            pl.BlockSpec((1, 1, 1), lambda b, n: (b, 0, 0), memory_space=pltpu.SMEM),
            pl.BlockSpec((1, 1, 1), lambda b, n: (b, 0, 0), memory_space=pltpu.SMEM),
        ],
        out_shape=[
            jax.ShapeDtypeStruct((B, 1, S), jnp.float32),
            jax.ShapeDtypeStruct((B, 1, S), jnp.float32),
            jax.ShapeDtypeStruct((B, 1, 1), jnp.float32),
            jax.ShapeDtypeStruct((B, 1, 1), jnp.float32),
        ],
    )(memory_matrix, lookup_keys3d, strengths3d, usage3d,
      write_prev3d, read_weights, free_gates3d)


def _sc_body(uu_hbm, alloc_hbm, keyA, keyB, payA, payB, alloc_v,
             hist, start, shf, shi):
    nc = 2
    wid = lax.axis_index("s") * nc + lax.axis_index("c")
    pltpu.sync_copy(uu_hbm.at[wid], keyA)

    ln = lax.iota(jnp.int32, 16)
    base_idx = ln * BLK
    ones_i = jnp.ones((16,), jnp.int32)
    zeros_i = jnp.zeros((16,), jnp.int32)
    shi[pl.ds(0, 16)] = zeros_i
    shf[pl.ds(0, 16)] = jnp.ones((16,), jnp.float32)
    bcast31 = jnp.full((16,), 31, jnp.int32)

    bufs = [(keyA, payA), (keyB, payB)]
    for p in range(4):
        sh = 8 * p
        ksrc, psrc = bufs[p % 2]
        kdst, pdst = bufs[(p + 1) % 2]

        def zero_body(i, _):
            hist[pl.ds(i * 16, 16)] = zeros_i
            return 0
        lax.fori_loop(0, 256, zero_body, 0)

        def cnt_body(t, _, ksrc=ksrc, sh=sh):
            k = plsc.load_gather(ksrc, [base_idx + t])
            d = jnp.bitwise_and(
                lax.shift_right_logical(plsc.bitcast(k, jnp.int32), sh), 255)
            plsc.addupdate_scatter(hist, [d * 16 + ln], ones_i)
            return 0
        lax.fori_loop(0, BLK, cnt_body, 0)

        def scan_body(i, carry):
            v = hist[pl.ds(i * 16, 16)]
            s = plsc.cumsum(v)
            shi[pl.ds(16, 16)] = s
            excl = plsc.load_gather(shi, [ln + 15])
            start[pl.ds(i * 16, 16)] = carry + excl
            tot = plsc.load_gather(shi, [bcast31])
            return carry + tot
        lax.fori_loop(0, 256, scan_body, zeros_i)

        def perm_body(t, _, ksrc=ksrc, psrc=psrc, kdst=kdst, pdst=pdst,
                      sh=sh, first=(p == 0)):
            eidx = base_idx + t
            k = plsc.load_gather(ksrc, [eidx])
            pay = eidx if first else plsc.load_gather(psrc, [eidx])
            d = jnp.bitwise_and(
                lax.shift_right_logical(plsc.bitcast(k, jnp.int32), sh), 255)
            hidx = d * 16 + ln
            off = plsc.load_gather(start, [hidx])
            plsc.store_scatter(kdst, [off], k)
            plsc.store_scatter(pdst, [off], pay)
            plsc.addupdate_scatter(start, [hidx], ones_i)
            return 0
        lax.fori_loop(0, BLK, perm_body, 0)

    def alloc_body(v, carry):
        c, deadf = carry
        su = keyA[pl.ds(v * 16, 16)]
        P = su
        for k in (1, 2, 4, 8):
            shf[pl.ds(16, 16)] = P
            g = plsc.load_gather(shf, [ln + (16 - k)])
            P = P * g
        shf[pl.ds(16, 16)] = P
        X = plsc.load_gather(shf, [ln + 15])
        E = c * X
        deadnow = jnp.where(E < STICKY_THR, 1.0, 0.0)
        deadcum = jnp.maximum(plsc.cummax(deadnow), deadf)
        E = jnp.where(deadcum > 0.0, 0.0, E)
        a = (1.0 - su) * E
        idx = payA[pl.ds(v * 16, 16)]
        plsc.store_scatter(alloc_v, [idx], a)
        Plast = plsc.load_gather(shf, [bcast31])
        dead_b = jnp.full((16,), jnp.max(deadcum), jnp.float32)
        c2 = jnp.where(dead_b > 0.0, 0.0, c * Plast)
        return (c2, dead_b)
    lax.fori_loop(0, S // 16, alloc_body,
                  (jnp.ones((16,), jnp.float32),
                   jnp.zeros((16,), jnp.float32)))

    pltpu.sync_copy(alloc_v, alloc_hbm.at[wid])


def _alloc_sc(uu):
    mesh = plsc.VectorSubcoreMesh(core_axis_name="c", subcore_axis_name="s")
    return pl.kernel(
        _sc_body,
        mesh=mesh,
        compiler_params=pltpu.CompilerParams(needs_layout_passes=False),
        out_type=jax.ShapeDtypeStruct((B, S), jnp.float32),
        scratch_types=[
            pltpu.VMEM((S,), jnp.float32),
            pltpu.VMEM((S,), jnp.float32),
            pltpu.VMEM((S,), jnp.int32),
            pltpu.VMEM((S,), jnp.int32),
            pltpu.VMEM((S,), jnp.float32),
            pltpu.VMEM((4096,), jnp.int32),
            pltpu.VMEM((4096,), jnp.int32),
            pltpu.VMEM((32,), jnp.float32),
            pltpu.VMEM((32,), jnp.int32),
        ],
    )(uu)


def _k3_body(mem_ref, z_ref, alloc_ref, rw_ref, m_ref, l_ref, wg_ref,
             ag_ref, wv_ref, ev_ref, nm_ref, rv_ref):
    ns = pl.program_id(1)
    z = z_ref[0, 0]
    lw = jnp.exp(z - m_ref[0, 0, 0]) / l_ref[0, 0, 0]
    ag = ag_ref[0, 0, 0]
    ww = wg_ref[0, 0, 0] * (ag * alloc_ref[0, 0] + (1.0 - ag) * lw)
    mem = mem_ref[0]
    ev = ev_ref[0, 0]
    wv = wv_ref[0, 0]
    nm = mem * (1.0 - ww[:, None] * ev[None, :]) + ww[:, None] * wv[None, :]
    nm_ref[0] = nm
    rv = lax.dot_general(nm, rw_ref[0], (((0,), (0,)), ((), ())),
                         preferred_element_type=jnp.float32)

    @pl.when(ns == 0)
    def _():
        rv_ref[0] = rv

    @pl.when(ns != 0)
    def _():
        rv_ref[0] += rv


def _k3_call(memory_matrix, z3d, alloc3d, read_weights, m3d, l3d, wg3d,
             ag3d, wv3d, ev3d):
    return pl.pallas_call(
        _k3_body,
        grid=(B, NSB),
        in_specs=[
            pl.BlockSpec((1, BS, D), lambda b, n: (b, n, 0)),
            pl.BlockSpec((1, 1, BS), lambda b, n: (b, 0, n)),
            pl.BlockSpec((1, 1, BS), lambda b, n: (b, 0, n)),
            pl.BlockSpec((1, BS, H), lambda b, n: (b, n, 0)),
            pl.BlockSpec((1, 1, 1), lambda b, n: (b, 0, 0), memory_space=pltpu.SMEM),
            pl.BlockSpec((1, 1, 1), lambda b, n: (b, 0, 0), memory_space=pltpu.SMEM),
            pl.BlockSpec((1, 1, 1), lambda b, n: (b, 0, 0), memory_space=pltpu.SMEM),
            pl.BlockSpec((1, 1, 1), lambda b, n: (b, 0, 0), memory_space=pltpu.SMEM),
            pl.BlockSpec((1, 1, D), lambda b, n: (b, 0, 0)),
            pl.BlockSpec((1, 1, D), lambda b, n: (b, 0, 0)),
        ],
        out_specs=[
            pl.BlockSpec((1, BS, D), lambda b, n: (b, n, 0)),
            pl.BlockSpec((1, D, H), lambda b, n: (b, 0, 0)),
        ],
        out_shape=[
            jax.ShapeDtypeStruct((B, S, D), jnp.float32),
            jax.ShapeDtypeStruct((B, D, H), jnp.float32),
        ],
    )(memory_matrix, z3d, alloc3d, read_weights, m3d, l3d, wg3d,
      ag3d, wv3d, ev3d)


def kernel(memory_matrix, usage_vector, read_weights, write_weight_prev,
           free_gates, lookup_keys, strengths, write_gate, allocation_gate,
           write_vector, erase_vector):
    r3 = lambda x: x.reshape(B, 1, -1)
    uu, z, m, l = _k1_call(
        memory_matrix, r3(lookup_keys), r3(strengths), r3(usage_vector),
        r3(write_weight_prev), read_weights, r3(free_gates))
    alloc = _alloc_sc(uu.reshape(B, S))
    new_memory, read_vecs = _k3_call(
        memory_matrix, z, r3(alloc), read_weights, m, l, r3(write_gate),
        r3(allocation_gate), r3(write_vector), r3(erase_vector))
    return new_memory, read_vecs


# speedup vs baseline: 4.5551x; 3.6103x over previous
import functools

import jax
import jax.numpy as jnp
from jax import lax
from jax.experimental import pallas as pl
from jax.experimental.pallas import tpu as pltpu
from jax.experimental.pallas import tpu_sc as plsc

B, S, D, H = 32, 16384, 64, 4
LANES = 16
BLK = S // LANES
BS = 4096
NSB = S // BS
EPS = 1e-8
STICKY_THR = 1e-35


def _k1_body(mem_ref, key_ref, str_ref, u_ref, w_ref, rw_ref, fg_ref,
             uu_ref, z_ref, m_ref, l_ref):
    ns = pl.program_id(1)
    mem = mem_ref[0]
    key = key_ref[0, 0]
    kn = key / (jnp.sqrt(jnp.sum(key * key)) + EPS)
    cosn = lax.dot_general(kn[None, :], mem, (((1,), (0,)), ((), ())),
                           preferred_element_type=jnp.float32)[0]
    nr = jnp.sqrt(jnp.sum(mem * mem, axis=0))
    cos = cosn / (nr + EPS)
    z = cos * str_ref[0, 0, 0]
    z_ref[0, 0] = z

    rw = rw_ref[0]
    ret = ((2.0 - rw[0] * fg_ref[0, 0, 0]) * (2.0 - rw[1] * fg_ref[0, 0, 1])
           * (2.0 - rw[2] * fg_ref[0, 0, 2]) * (2.0 - rw[3] * fg_ref[0, 0, 3]))
    u = u_ref[0, 0]
    w = w_ref[0, 0]
    uu_ref[0, 0] = (u + w - u * w) * ret

    bm = jnp.max(z)
    bl = jnp.sum(jnp.exp(z - bm))

    @pl.when(ns == 0)
    def _():
        m_ref[0, 0, 0] = bm
        l_ref[0, 0, 0] = bl

    @pl.when(ns != 0)
    def _():
        m_prev = m_ref[0, 0, 0]
        l_prev = l_ref[0, 0, 0]
        m_new = jnp.maximum(m_prev, bm)
        l_ref[0, 0, 0] = (l_prev * jnp.exp(m_prev - m_new)
                          + bl * jnp.exp(bm - m_new))
        m_ref[0, 0, 0] = m_new


def _k1_call(memory_matrix, lookup_keys3d, strengths3d, usage3d,
             write_prev3d, read_weights, free_gates3d):
    return pl.pallas_call(
        _k1_body,
        grid=(B, NSB),
        in_specs=[
            pl.BlockSpec((1, D, BS), lambda b, n: (b, 0, n)),
            pl.BlockSpec((1, 1, D), lambda b, n: (b, 0, 0)),
            pl.BlockSpec((1, 1, 1), lambda b, n: (b, 0, 0), memory_space=pltpu.SMEM),
            pl.BlockSpec((1, 1, BS), lambda b, n: (b, 0, n)),
            pl.BlockSpec((1, 1, BS), lambda b, n: (b, 0, n)),
            pl.BlockSpec((1, H, BS), lambda b, n: (b, 0, n)),
            pl.BlockSpec((1, 1, H), lambda b, n: (b, 0, 0), memory_space=pltpu.SMEM),
        ],
        out_specs=[
            pl.BlockSpec((1, 1, BS), lambda b, n: (b, 0, n)),
            pl.BlockSpec((1, 1, BS), lambda b, n: (b, 0, n)),
            pl.BlockSpec((1, 1, 1), lambda b, n: (b, 0, 0), memory_space=pltpu.SMEM),
            pl.BlockSpec((1, 1, 1), lambda b, n: (b, 0, 0), memory_space=pltpu.SMEM),
        ],
        out_shape=[
            jax.ShapeDtypeStruct((B, 1, S), jnp.float32),
            jax.ShapeDtypeStruct((B, 1, S), jnp.float32),
            jax.ShapeDtypeStruct((B, 1, 1), jnp.float32),
            jax.ShapeDtypeStruct((B, 1, 1), jnp.float32),
        ],
    )(memory_matrix, lookup_keys3d, strengths3d, usage3d,
      write_prev3d, read_weights, free_gates3d)


def _sc_body(uu_hbm, alloc_hbm, keyA, keyB, payA, payB, alloc_v,
             hist, start, shf, shi):
    nc = 2
    wid = lax.axis_index("s") * nc + lax.axis_index("c")
    pltpu.sync_copy(uu_hbm.at[wid], keyA)

    ln = lax.iota(jnp.int32, 16)
    base_idx = ln * BLK
    ones_i = jnp.ones((16,), jnp.int32)
    zeros_i = jnp.zeros((16,), jnp.int32)
    shi[pl.ds(0, 16)] = zeros_i
    shf[pl.ds(0, 16)] = jnp.ones((16,), jnp.float32)
    bcast31 = jnp.full((16,), 31, jnp.int32)

    bufs = [(keyA, payA), (keyB, payB)]
    for p in range(4):
        sh = 8 * p
        ksrc, psrc = bufs[p % 2]
        kdst, pdst = bufs[(p + 1) % 2]

        def zero_body(i, _):
            hist[pl.ds(i * 16, 16)] = zeros_i
            return 0
        lax.fori_loop(0, 256, zero_body, 0)

        def cnt_body(t, _, ksrc=ksrc, sh=sh):
            k = plsc.load_gather(ksrc, [base_idx + t])
            d = jnp.bitwise_and(
                lax.shift_right_logical(plsc.bitcast(k, jnp.int32), sh), 255)
            plsc.addupdate_scatter(hist, [d * 16 + ln], ones_i)
            return 0
        lax.fori_loop(0, BLK, cnt_body, 0)

        def scan_body(i, carry):
            v = hist[pl.ds(i * 16, 16)]
            s = plsc.cumsum(v)
            shi[pl.ds(16, 16)] = s
            excl = plsc.load_gather(shi, [ln + 15])
            start[pl.ds(i * 16, 16)] = carry + excl
            tot = plsc.load_gather(shi, [bcast31])
            return carry + tot
        lax.fori_loop(0, 256, scan_body, zeros_i)

        def perm_body(t, _, ksrc=ksrc, psrc=psrc, kdst=kdst, pdst=pdst,
                      sh=sh, first=(p == 0)):
            eidx = base_idx + t
            k = plsc.load_gather(ksrc, [eidx])
            pay = eidx if first else plsc.load_gather(psrc, [eidx])
            d = jnp.bitwise_and(
                lax.shift_right_logical(plsc.bitcast(k, jnp.int32), sh), 255)
            hidx = d * 16 + ln
            off = plsc.load_gather(start, [hidx])
            plsc.store_scatter(kdst, [off], k)
            plsc.store_scatter(pdst, [off], pay)
            plsc.addupdate_scatter(start, [hidx], ones_i)
            return 0
        lax.fori_loop(0, BLK, perm_body, 0)

    def alloc_body(v, carry):
        c, deadf = carry
        su = keyA[pl.ds(v * 16, 16)]
        P = su
        for k in (1, 2, 4, 8):
            shf[pl.ds(16, 16)] = P
            g = plsc.load_gather(shf, [ln + (16 - k)])
            P = P * g
        shf[pl.ds(16, 16)] = P
        X = plsc.load_gather(shf, [ln + 15])
        E = c * X
        deadnow = jnp.where(E < STICKY_THR, 1.0, 0.0)
        deadcum = jnp.maximum(plsc.cummax(deadnow), deadf)
        E = jnp.where(deadcum > 0.0, 0.0, E)
        a = (1.0 - su) * E
        idx = payA[pl.ds(v * 16, 16)]
        plsc.store_scatter(alloc_v, [idx], a)
        Plast = plsc.load_gather(shf, [bcast31])
        dead_b = jnp.full((16,), jnp.max(deadcum), jnp.float32)
        c2 = jnp.where(dead_b > 0.0, 0.0, c * Plast)
        return (c2, dead_b)
    lax.fori_loop(0, S // 16, alloc_body,
                  (jnp.ones((16,), jnp.float32),
                   jnp.zeros((16,), jnp.float32)))

    pltpu.sync_copy(alloc_v, alloc_hbm.at[wid])


def _alloc_sc(uu):
    mesh = plsc.VectorSubcoreMesh(core_axis_name="c", subcore_axis_name="s")
    return pl.kernel(
        _sc_body,
        mesh=mesh,
        compiler_params=pltpu.CompilerParams(needs_layout_passes=False),
        out_type=jax.ShapeDtypeStruct((B, S), jnp.float32),
        scratch_types=[
            pltpu.VMEM((S,), jnp.float32),
            pltpu.VMEM((S,), jnp.float32),
            pltpu.VMEM((S,), jnp.int32),
            pltpu.VMEM((S,), jnp.int32),
            pltpu.VMEM((S,), jnp.float32),
            pltpu.VMEM((4096,), jnp.int32),
            pltpu.VMEM((4096,), jnp.int32),
            pltpu.VMEM((32,), jnp.float32),
            pltpu.VMEM((32,), jnp.int32),
        ],
    )(uu)


def _k3_body(mem_ref, z_ref, alloc_ref, rw_ref, m_ref, l_ref, wg_ref,
             ag_ref, wv_ref, ev_ref, nm_ref, rv_ref):
    ns = pl.program_id(1)
    z = z_ref[0, 0]
    lw = jnp.exp(z - m_ref[0, 0, 0]) / l_ref[0, 0, 0]
    ag = ag_ref[0, 0, 0]
    ww = wg_ref[0, 0, 0] * (ag * alloc_ref[0, 0] + (1.0 - ag) * lw)
    mem = mem_ref[0]
    ev = ev_ref[0, :, 0]
    wv = wv_ref[0, :, 0]
    nm = (mem * (1.0 - ev[:, None] * ww[None, :])
          + wv[:, None] * ww[None, :])
    nm_ref[0] = nm
    rv = lax.dot_general(nm, rw_ref[0], (((1,), (1,)), ((), ())),
                         preferred_element_type=jnp.float32)

    @pl.when(ns == 0)
    def _():
        rv_ref[0] = rv

    @pl.when(ns != 0)
    def _():
        rv_ref[0] += rv


def _k3_call(memory_matrix, z3d, alloc3d, read_weights, m3d, l3d, wg3d,
             ag3d, wv3d, ev3d):
    return pl.pallas_call(
        _k3_body,
        grid=(B, NSB),
        in_specs=[
            pl.BlockSpec((1, D, BS), lambda b, n: (b, 0, n)),
            pl.BlockSpec((1, 1, BS), lambda b, n: (b, 0, n)),
            pl.BlockSpec((1, 1, BS), lambda b, n: (b, 0, n)),
            pl.BlockSpec((1, H, BS), lambda b, n: (b, 0, n)),
            pl.BlockSpec((1, 1, 1), lambda b, n: (b, 0, 0), memory_space=pltpu.SMEM),
            pl.BlockSpec((1, 1, 1), lambda b, n: (b, 0, 0), memory_space=pltpu.SMEM),
            pl.BlockSpec((1, 1, 1), lambda b, n: (b, 0, 0), memory_space=pltpu.SMEM),
            pl.BlockSpec((1, 1, 1), lambda b, n: (b, 0, 0), memory_space=pltpu.SMEM),
            pl.BlockSpec((1, D, 1), lambda b, n: (b, 0, 0)),
            pl.BlockSpec((1, D, 1), lambda b, n: (b, 0, 0)),
        ],
        out_specs=[
            pl.BlockSpec((1, D, BS), lambda b, n: (b, 0, n)),
            pl.BlockSpec((1, D, H), lambda b, n: (b, 0, 0)),
        ],
        out_shape=[
            jax.ShapeDtypeStruct((B, D, S), jnp.float32),
            jax.ShapeDtypeStruct((B, D, H), jnp.float32),
        ],
    )(memory_matrix, z3d, alloc3d, read_weights, m3d, l3d, wg3d,
      ag3d, wv3d, ev3d)


def kernel(memory_matrix, usage_vector, read_weights, write_weight_prev,
           free_gates, lookup_keys, strengths, write_gate, allocation_gate,
           write_vector, erase_vector):
    r3 = lambda x: x.reshape(B, 1, -1)
    mem_t = jnp.swapaxes(memory_matrix, 1, 2)
    rw_t = jnp.swapaxes(read_weights, 1, 2)
    uu, z, m, l = _k1_call(
        mem_t, r3(lookup_keys), r3(strengths), r3(usage_vector),
        r3(write_weight_prev), rw_t, r3(free_gates))
    alloc = _alloc_sc(uu.reshape(B, S))
    new_mem_t, read_vecs = _k3_call(
        mem_t, z, r3(alloc), rw_t, m, l, r3(write_gate),
        r3(allocation_gate), write_vector.reshape(B, D, 1),
        erase_vector.reshape(B, D, 1))
    return jnp.swapaxes(new_mem_t, 1, 2), read_vecs


# speedup vs baseline: 6.6079x; 1.4506x over previous
import functools

import jax
import jax.numpy as jnp
from jax import lax
from jax.experimental import pallas as pl
from jax.experimental.pallas import tpu as pltpu
from jax.experimental.pallas import tpu_sc as plsc

B, S, D, H = 32, 16384, 64, 4
LANES = 16
BLK = S // LANES
BS = 16384
NSB = S // BS
EPS = 1e-8
STICKY_THR = 1e-35


def _k0_body(u_ref, w_ref, rw_ref, fg_ref, uu_ref):
    rw = rw_ref[0]
    ret = ((2.0 - rw[0] * fg_ref[0, 0, 0]) * (2.0 - rw[1] * fg_ref[0, 0, 1])
           * (2.0 - rw[2] * fg_ref[0, 0, 2]) * (2.0 - rw[3] * fg_ref[0, 0, 3]))
    u = u_ref[0, 0]
    w = w_ref[0, 0]
    uu_ref[0, 0] = (u + w - u * w) * ret


def _k0_call(usage3d, write_prev3d, read_weights, free_gates3d):
    return pl.pallas_call(
        _k0_body,
        grid=(B, NSB),
        in_specs=[
            pl.BlockSpec((1, 1, BS), lambda b, n: (b, 0, n)),
            pl.BlockSpec((1, 1, BS), lambda b, n: (b, 0, n)),
            pl.BlockSpec((1, H, BS), lambda b, n: (b, 0, n)),
            pl.BlockSpec((1, 1, H), lambda b, n: (b, 0, 0), memory_space=pltpu.SMEM),
        ],
        out_specs=[
            pl.BlockSpec((1, 1, BS), lambda b, n: (b, 0, n)),
        ],
        out_shape=[
            jax.ShapeDtypeStruct((B, 1, S), jnp.float32),
        ],
    )(usage3d, write_prev3d, read_weights, free_gates3d)


def _k1_body(mem_ref, key_ref, str_ref, z_ref, m_ref, l_ref):
    ns = pl.program_id(1)
    mem = mem_ref[0]
    key = key_ref[0, 0]
    kn = key / (jnp.sqrt(jnp.sum(key * key)) + EPS)
    cosn = lax.dot_general(kn[None, :], mem, (((1,), (0,)), ((), ())),
                           preferred_element_type=jnp.float32)[0]
    nr = jnp.sqrt(jnp.sum(mem * mem, axis=0))
    cos = cosn / (nr + EPS)
    z = cos * str_ref[0, 0, 0]
    z_ref[0, 0] = z

    bm = jnp.max(z)
    bl = jnp.sum(jnp.exp(z - bm))

    @pl.when(ns == 0)
    def _():
        m_ref[0, 0, 0] = bm
        l_ref[0, 0, 0] = bl

    @pl.when(ns != 0)
    def _():
        m_prev = m_ref[0, 0, 0]
        l_prev = l_ref[0, 0, 0]
        m_new = jnp.maximum(m_prev, bm)
        l_ref[0, 0, 0] = (l_prev * jnp.exp(m_prev - m_new)
                          + bl * jnp.exp(bm - m_new))
        m_ref[0, 0, 0] = m_new


def _k1_call(memory_matrix, lookup_keys3d, strengths3d):
    return pl.pallas_call(
        _k1_body,
        grid=(B, NSB),
        in_specs=[
            pl.BlockSpec((1, D, BS), lambda b, n: (b, 0, n)),
            pl.BlockSpec((1, 1, D), lambda b, n: (b, 0, 0)),
            pl.BlockSpec((1, 1, 1), lambda b, n: (b, 0, 0), memory_space=pltpu.SMEM),
        ],
        out_specs=[
            pl.BlockSpec((1, 1, BS), lambda b, n: (b, 0, n)),
            pl.BlockSpec((1, 1, 1), lambda b, n: (b, 0, 0), memory_space=pltpu.SMEM),
            pl.BlockSpec((1, 1, 1), lambda b, n: (b, 0, 0), memory_space=pltpu.SMEM),
        ],
        out_shape=[
            jax.ShapeDtypeStruct((B, 1, S), jnp.float32),
            jax.ShapeDtypeStruct((B, 1, 1), jnp.float32),
            jax.ShapeDtypeStruct((B, 1, 1), jnp.float32),
        ],
    )(memory_matrix, lookup_keys3d, strengths3d)


def _sc_body(uu_hbm, alloc_hbm, keyA, keyB, payA, payB, alloc_v,
             hist, start, shf, shi):
    nc = 2
    wid = lax.axis_index("s") * nc + lax.axis_index("c")
    pltpu.sync_copy(uu_hbm.at[wid], keyA)

    ln = lax.iota(jnp.int32, 16)
    base_idx = ln * BLK
    ones_i = jnp.ones((16,), jnp.int32)
    zeros_i = jnp.zeros((16,), jnp.int32)
    shi[pl.ds(0, 16)] = zeros_i
    shf[pl.ds(0, 16)] = jnp.ones((16,), jnp.float32)
    bcast31 = jnp.full((16,), 31, jnp.int32)

    bufs = [(keyA, payA), (keyB, payB)]
    for p in range(4):
        sh = 8 * p
        ksrc, psrc = bufs[p % 2]
        kdst, pdst = bufs[(p + 1) % 2]

        def zero_body(i, _):
            for j in range(4):
                hist[pl.ds((i * 4 + j) * 16, 16)] = zeros_i
            return 0
        lax.fori_loop(0, 64, zero_body, 0)

        def cnt_body(i, _, ksrc=ksrc, sh=sh):
            for j in range(8):
                t = i * 8 + j
                k = plsc.load_gather(ksrc, [base_idx + t])
                d = jnp.bitwise_and(
                    lax.shift_right_logical(plsc.bitcast(k, jnp.int32), sh),
                    255)
                plsc.addupdate_scatter(hist, [d * 16 + ln], ones_i)
            return 0
        lax.fori_loop(0, BLK // 8, cnt_body, 0)

        def scan_body(i, carry):
            v = hist[pl.ds(i * 16, 16)]
            s = plsc.cumsum(v)
            shi[pl.ds(16, 16)] = s
            excl = plsc.load_gather(shi, [ln + 15])
            start[pl.ds(i * 16, 16)] = carry + excl
            tot = plsc.load_gather(shi, [bcast31])
            return carry + tot
        lax.fori_loop(0, 256, scan_body, zeros_i)

        def perm_body(i, _, ksrc=ksrc, psrc=psrc, kdst=kdst, pdst=pdst,
                      sh=sh, first=(p == 0)):
            for j in range(4):
                t = i * 4 + j
                eidx = base_idx + t
                k = plsc.load_gather(ksrc, [eidx])
                pay = eidx if first else plsc.load_gather(psrc, [eidx])
                d = jnp.bitwise_and(
                    lax.shift_right_logical(plsc.bitcast(k, jnp.int32), sh),
                    255)
                hidx = d * 16 + ln
                off = plsc.load_gather(start, [hidx])
                plsc.store_scatter(kdst, [off], k)
                plsc.store_scatter(pdst, [off], pay)
                plsc.addupdate_scatter(start, [hidx], ones_i)
            return 0
        lax.fori_loop(0, BLK // 4, perm_body, 0)

    def alloc_body(v, carry):
        c, deadf = carry
        su = keyA[pl.ds(v * 16, 16)]
        P = su
        for k in (1, 2, 4, 8):
            shf[pl.ds(16, 16)] = P
            g = plsc.load_gather(shf, [ln + (16 - k)])
            P = P * g
        shf[pl.ds(16, 16)] = P
        X = plsc.load_gather(shf, [ln + 15])
        E = c * X
        deadnow = jnp.where(E < STICKY_THR, 1.0, 0.0)
        deadcum = jnp.maximum(plsc.cummax(deadnow), deadf)
        E = jnp.where(deadcum > 0.0, 0.0, E)
        a = (1.0 - su) * E
        idx = payA[pl.ds(v * 16, 16)]
        plsc.store_scatter(alloc_v, [idx], a)
        Plast = plsc.load_gather(shf, [bcast31])
        dead_b = jnp.full((16,), jnp.max(deadcum), jnp.float32)
        c2 = jnp.where(dead_b > 0.0, 0.0, c * Plast)
        return (c2, dead_b)
    lax.fori_loop(0, S // 16, alloc_body,
                  (jnp.ones((16,), jnp.float32),
                   jnp.zeros((16,), jnp.float32)))

    pltpu.sync_copy(alloc_v, alloc_hbm.at[wid])


def _alloc_sc(uu):
    mesh = plsc.VectorSubcoreMesh(core_axis_name="c", subcore_axis_name="s")
    return pl.kernel(
        _sc_body,
        mesh=mesh,
        compiler_params=pltpu.CompilerParams(needs_layout_passes=False),
        out_type=jax.ShapeDtypeStruct((B, S), jnp.float32),
        scratch_types=[
            pltpu.VMEM((S,), jnp.float32),
            pltpu.VMEM((S,), jnp.float32),
            pltpu.VMEM((S,), jnp.int32),
            pltpu.VMEM((S,), jnp.int32),
            pltpu.VMEM((S,), jnp.float32),
            pltpu.VMEM((4096,), jnp.int32),
            pltpu.VMEM((4096,), jnp.int32),
            pltpu.VMEM((32,), jnp.float32),
            pltpu.VMEM((32,), jnp.int32),
        ],
    )(uu)


def _k3_body(mem_ref, z_ref, alloc_ref, rw_ref, m_ref, l_ref, wg_ref,
             ag_ref, wv_ref, ev_ref, nm_ref, rv_ref):
    ns = pl.program_id(1)
    z = z_ref[0, 0]
    lw = jnp.exp(z - m_ref[0, 0, 0]) / l_ref[0, 0, 0]
    ag = ag_ref[0, 0, 0]
    ww = wg_ref[0, 0, 0] * (ag * alloc_ref[0, 0] + (1.0 - ag) * lw)
    mem = mem_ref[0]
    ev = ev_ref[0, :, 0]
    wv = wv_ref[0, :, 0]
    nm = (mem * (1.0 - ev[:, None] * ww[None, :])
          + wv[:, None] * ww[None, :])
    nm_ref[0] = nm
    rv = lax.dot_general(nm, rw_ref[0], (((1,), (1,)), ((), ())),
                         preferred_element_type=jnp.float32)

    @pl.when(ns == 0)
    def _():
        rv_ref[0] = rv

    @pl.when(ns != 0)
    def _():
        rv_ref[0] += rv


def _k3_call(memory_matrix, z3d, alloc3d, read_weights, m3d, l3d, wg3d,
             ag3d, wv3d, ev3d):
    return pl.pallas_call(
        _k3_body,
        grid=(B, NSB),
        in_specs=[
            pl.BlockSpec((1, D, BS), lambda b, n: (b, 0, n)),
            pl.BlockSpec((1, 1, BS), lambda b, n: (b, 0, n)),
            pl.BlockSpec((1, 1, BS), lambda b, n: (b, 0, n)),
            pl.BlockSpec((1, H, BS), lambda b, n: (b, 0, n)),
            pl.BlockSpec((1, 1, 1), lambda b, n: (b, 0, 0), memory_space=pltpu.SMEM),
            pl.BlockSpec((1, 1, 1), lambda b, n: (b, 0, 0), memory_space=pltpu.SMEM),
            pl.BlockSpec((1, 1, 1), lambda b, n: (b, 0, 0), memory_space=pltpu.SMEM),
            pl.BlockSpec((1, 1, 1), lambda b, n: (b, 0, 0), memory_space=pltpu.SMEM),
            pl.BlockSpec((1, D, 1), lambda b, n: (b, 0, 0)),
            pl.BlockSpec((1, D, 1), lambda b, n: (b, 0, 0)),
        ],
        out_specs=[
            pl.BlockSpec((1, D, BS), lambda b, n: (b, 0, n)),
            pl.BlockSpec((1, D, H), lambda b, n: (b, 0, 0)),
        ],
        out_shape=[
            jax.ShapeDtypeStruct((B, D, S), jnp.float32),
            jax.ShapeDtypeStruct((B, D, H), jnp.float32),
        ],
    )(memory_matrix, z3d, alloc3d, read_weights, m3d, l3d, wg3d,
      ag3d, wv3d, ev3d)


def kernel(memory_matrix, usage_vector, read_weights, write_weight_prev,
           free_gates, lookup_keys, strengths, write_gate, allocation_gate,
           write_vector, erase_vector):
    r3 = lambda x: x.reshape(B, 1, -1)
    mem_t = jnp.swapaxes(memory_matrix, 1, 2)
    rw_t = jnp.swapaxes(read_weights, 1, 2)
    uu, = _k0_call(r3(usage_vector), r3(write_weight_prev), rw_t,
                   r3(free_gates))
    alloc = _alloc_sc(uu.reshape(B, S))
    z, m, l = _k1_call(mem_t, r3(lookup_keys), r3(strengths))
    new_mem_t, read_vecs = _k3_call(
        mem_t, z, r3(alloc), rw_t, m, l, r3(write_gate),
        r3(allocation_gate), write_vector.reshape(B, D, 1),
        erase_vector.reshape(B, D, 1))
    return jnp.swapaxes(new_mem_t, 1, 2), read_vecs


# speedup vs baseline: 8.6597x; 1.3105x over previous
import functools

import jax
import jax.numpy as jnp
from jax import lax
from jax.experimental import pallas as pl
from jax.experimental.pallas import tpu as pltpu
from jax.experimental.pallas import tpu_sc as plsc

B, S, D, H = 32, 16384, 64, 4
LANES = 16
BLK = S // LANES
BS = 16384
NSB = S // BS
EPS = 1e-8
STICKY_THR = 1e-35


def _k0_body(u_ref, w_ref, rw_ref, fg_ref, uu_ref):
    rw = rw_ref[0]
    ret = ((2.0 - rw[0] * fg_ref[0, 0, 0]) * (2.0 - rw[1] * fg_ref[0, 0, 1])
           * (2.0 - rw[2] * fg_ref[0, 0, 2]) * (2.0 - rw[3] * fg_ref[0, 0, 3]))
    u = u_ref[0, 0]
    w = w_ref[0, 0]
    uu_ref[0, 0] = (u + w - u * w) * ret


def _k0_call(usage3d, write_prev3d, read_weights, free_gates3d):
    return pl.pallas_call(
        _k0_body,
        grid=(B, NSB),
        in_specs=[
            pl.BlockSpec((1, 1, BS), lambda b, n: (b, 0, n)),
            pl.BlockSpec((1, 1, BS), lambda b, n: (b, 0, n)),
            pl.BlockSpec((1, H, BS), lambda b, n: (b, 0, n)),
            pl.BlockSpec((1, 1, H), lambda b, n: (b, 0, 0), memory_space=pltpu.SMEM),
        ],
        out_specs=[
            pl.BlockSpec((1, 1, BS), lambda b, n: (b, 0, n)),
        ],
        out_shape=[
            jax.ShapeDtypeStruct((B, 1, S), jnp.float32),
        ],
    )(usage3d, write_prev3d, read_weights, free_gates3d)


def _k1_body(mem_ref, key_ref, str_ref, z_ref, m_ref, l_ref):
    ns = pl.program_id(1)
    mem = mem_ref[0]
    key = key_ref[0, 0]
    kn = key / (jnp.sqrt(jnp.sum(key * key)) + EPS)
    cosn = lax.dot_general(kn[None, :], mem, (((1,), (0,)), ((), ())),
                           preferred_element_type=jnp.float32)[0]
    nr = jnp.sqrt(jnp.sum(mem * mem, axis=0))
    cos = cosn / (nr + EPS)
    z = cos * str_ref[0, 0, 0]
    z_ref[0, 0] = z

    bm = jnp.max(z)
    bl = jnp.sum(jnp.exp(z - bm))

    @pl.when(ns == 0)
    def _():
        m_ref[0, 0, 0] = bm
        l_ref[0, 0, 0] = bl

    @pl.when(ns != 0)
    def _():
        m_prev = m_ref[0, 0, 0]
        l_prev = l_ref[0, 0, 0]
        m_new = jnp.maximum(m_prev, bm)
        l_ref[0, 0, 0] = (l_prev * jnp.exp(m_prev - m_new)
                          + bl * jnp.exp(bm - m_new))
        m_ref[0, 0, 0] = m_new


def _k1_call(memory_matrix, lookup_keys3d, strengths3d):
    return pl.pallas_call(
        _k1_body,
        grid=(B, NSB),
        in_specs=[
            pl.BlockSpec((1, D, BS), lambda b, n: (b, 0, n)),
            pl.BlockSpec((1, 1, D), lambda b, n: (b, 0, 0)),
            pl.BlockSpec((1, 1, 1), lambda b, n: (b, 0, 0), memory_space=pltpu.SMEM),
        ],
        out_specs=[
            pl.BlockSpec((1, 1, BS), lambda b, n: (b, 0, n)),
            pl.BlockSpec((1, 1, 1), lambda b, n: (b, 0, 0), memory_space=pltpu.SMEM),
            pl.BlockSpec((1, 1, 1), lambda b, n: (b, 0, 0), memory_space=pltpu.SMEM),
        ],
        out_shape=[
            jax.ShapeDtypeStruct((B, 1, S), jnp.float32),
            jax.ShapeDtypeStruct((B, 1, 1), jnp.float32),
            jax.ShapeDtypeStruct((B, 1, 1), jnp.float32),
        ],
    )(memory_matrix, lookup_keys3d, strengths3d)


_LGC = (-2.4825606616e-02, 2.6685882287e-01, -1.2342631731e+00,
        3.2188328372e+00, -5.2641104772e+00, 6.0658301432e+00,
        -3.0283174811e+00)
_LN2 = 0.6931471805599453
_LOG2_STICKY = -116.27


def _sc_body(uu_hbm, alloc_hbm, keyA, keyB, payA, payB, alloc_v,
             hist, start, shf, shi):
    nc = 2
    wid = lax.axis_index("s") * nc + lax.axis_index("c")
    pltpu.sync_copy(uu_hbm.at[wid], keyA)

    ln = lax.iota(jnp.int32, 16)
    ones_i = jnp.ones((16,), jnp.int32)
    zeros_i = jnp.zeros((16,), jnp.int32)
    shi[pl.ds(0, 16)] = zeros_i
    bcast31 = jnp.full((16,), 31, jnp.int32)

    bufs = [(keyA, payA), (keyB, payB)]
    for p in range(4):
        sh = 8 * p
        ksrc, psrc = bufs[p % 2]
        kdst, pdst = bufs[(p + 1) % 2]
        last = (p == 3)

        def zero_body(i, _):
            for j in range(4):
                hist[pl.ds((i * 4 + j) * 16, 16)] = zeros_i
            return 0
        lax.fori_loop(0, 64, zero_body, 0)

        def cnt_body(i, _, ksrc=ksrc, sh=sh):
            for j in range(8):
                t = i * 8 + j
                k = ksrc[pl.ds(t * 16, 16)]
                d = jnp.bitwise_and(
                    lax.shift_right_logical(plsc.bitcast(k, jnp.int32), sh),
                    255)
                plsc.addupdate_scatter(hist, [d * 16 + ln], ones_i)
            return 0
        lax.fori_loop(0, BLK // 8, cnt_body, 0)

        def scan_body(i, carry):
            v = hist[pl.ds(i * 16, 16)]
            s = plsc.cumsum(v)
            shi[pl.ds(16, 16)] = s
            excl = plsc.load_gather(shi, [ln + 15])
            start[pl.ds(i * 16, 16)] = carry + excl
            tot = plsc.load_gather(shi, [bcast31])
            return carry + tot
        lax.fori_loop(0, 256, scan_body, zeros_i)

        def perm_body(i, _, ksrc=ksrc, psrc=psrc, kdst=kdst, pdst=pdst,
                      sh=sh, first=(p == 0), last=last):
            for j in range(4):
                t = i * 4 + j
                k = ksrc[pl.ds(t * 16, 16)]
                pay = (t * 16 + ln) if first else psrc[pl.ds(t * 16, 16)]
                d = jnp.bitwise_and(
                    lax.shift_right_logical(plsc.bitcast(k, jnp.int32), sh),
                    255)
                hidx = d * 16 + ln
                off = plsc.load_gather(start, [hidx])
                if last:
                    waddr = off
                else:
                    waddr = jnp.bitwise_or(
                        lax.shift_left(jnp.bitwise_and(off, 1023), 4),
                        lax.shift_right_logical(off, 10))
                plsc.store_scatter(kdst, [waddr], k)
                plsc.store_scatter(pdst, [waddr], pay)
                plsc.addupdate_scatter(start, [hidx], ones_i)
            return 0
        lax.fori_loop(0, BLK // 4, perm_body, 0)

    def alloc_body(v, carry):
        c, deadf = carry
        su = keyA[pl.ds(v * 16, 16)]
        ki = plsc.bitcast(su, jnp.int32)
        ef = (lax.shift_right_logical(ki, 23) - 127).astype(jnp.float32)
        m = plsc.bitcast(
            jnp.bitwise_or(jnp.bitwise_and(ki, 0x7FFFFF), 0x3F800000),
            jnp.float32)
        lg = jnp.float32(_LGC[0])
        for cc in _LGC[1:]:
            lg = lg * m + jnp.float32(cc)
        lg = lg + ef
        incl = plsc.cumsum(lg)
        y = c + (incl - lg)
        deadnow = jnp.where(y < _LOG2_STICKY, 1.0, 0.0)
        deadcum = jnp.maximum(plsc.cummax(deadnow), deadf)
        E = jnp.where(deadcum > 0.0, 0.0, jnp.exp(y * _LN2))
        a = (1.0 - su) * E
        pay = payA[pl.ds(v * 16, 16)]
        plsc.store_scatter(alloc_v, [pay], a)
        shf[pl.ds(16, 16)] = incl
        last_incl = plsc.load_gather(shf, [bcast31])
        dead_b = jnp.full((16,), jnp.max(deadcum), jnp.float32)
        return (c + last_incl, dead_b)
    lax.fori_loop(0, S // 16, alloc_body,
                  (jnp.zeros((16,), jnp.float32),
                   jnp.zeros((16,), jnp.float32)))

    pltpu.sync_copy(alloc_v, alloc_hbm.at[wid])


def _alloc_sc(uu):
    mesh = plsc.VectorSubcoreMesh(core_axis_name="c", subcore_axis_name="s")
    return pl.kernel(
        _sc_body,
        mesh=mesh,
        compiler_params=pltpu.CompilerParams(needs_layout_passes=False),
        out_type=jax.ShapeDtypeStruct((B, S), jnp.float32),
        scratch_types=[
            pltpu.VMEM((S,), jnp.float32),
            pltpu.VMEM((S,), jnp.float32),
            pltpu.VMEM((S,), jnp.int32),
            pltpu.VMEM((S,), jnp.int32),
            pltpu.VMEM((S,), jnp.float32),
            pltpu.VMEM((4096,), jnp.int32),
            pltpu.VMEM((4096,), jnp.int32),
            pltpu.VMEM((32,), jnp.float32),
            pltpu.VMEM((32,), jnp.int32),
        ],
    )(uu)


def _k3_body(mem_ref, z_ref, alloc_ref, rw_ref, m_ref, l_ref, wg_ref,
             ag_ref, wv_ref, ev_ref, nm_ref, rv_ref):
    ns = pl.program_id(1)
    z = z_ref[0, 0]
    lw = jnp.exp(z - m_ref[0, 0, 0]) / l_ref[0, 0, 0]
    ag = ag_ref[0, 0, 0]
    ww = wg_ref[0, 0, 0] * (ag * alloc_ref[0, 0] + (1.0 - ag) * lw)
    mem = mem_ref[0]
    ev = ev_ref[0, :, 0]
    wv = wv_ref[0, :, 0]
    nm = (mem * (1.0 - ev[:, None] * ww[None, :])
          + wv[:, None] * ww[None, :])
    nm_ref[0] = nm
    rv = lax.dot_general(nm, rw_ref[0], (((1,), (1,)), ((), ())),
                         preferred_element_type=jnp.float32)

    @pl.when(ns == 0)
    def _():
        rv_ref[0] = rv

    @pl.when(ns != 0)
    def _():
        rv_ref[0] += rv


def _k3_call(memory_matrix, z3d, alloc3d, read_weights, m3d, l3d, wg3d,
             ag3d, wv3d, ev3d):
    return pl.pallas_call(
        _k3_body,
        grid=(B, NSB),
        in_specs=[
            pl.BlockSpec((1, D, BS), lambda b, n: (b, 0, n)),
            pl.BlockSpec((1, 1, BS), lambda b, n: (b, 0, n)),
            pl.BlockSpec((1, 1, BS), lambda b, n: (b, 0, n)),
            pl.BlockSpec((1, H, BS), lambda b, n: (b, 0, n)),
            pl.BlockSpec((1, 1, 1), lambda b, n: (b, 0, 0), memory_space=pltpu.SMEM),
            pl.BlockSpec((1, 1, 1), lambda b, n: (b, 0, 0), memory_space=pltpu.SMEM),
            pl.BlockSpec((1, 1, 1), lambda b, n: (b, 0, 0), memory_space=pltpu.SMEM),
            pl.BlockSpec((1, 1, 1), lambda b, n: (b, 0, 0), memory_space=pltpu.SMEM),
            pl.BlockSpec((1, D, 1), lambda b, n: (b, 0, 0)),
            pl.BlockSpec((1, D, 1), lambda b, n: (b, 0, 0)),
        ],
        out_specs=[
            pl.BlockSpec((1, D, BS), lambda b, n: (b, 0, n)),
            pl.BlockSpec((1, D, H), lambda b, n: (b, 0, 0)),
        ],
        out_shape=[
            jax.ShapeDtypeStruct((B, D, S), jnp.float32),
            jax.ShapeDtypeStruct((B, D, H), jnp.float32),
        ],
    )(memory_matrix, z3d, alloc3d, read_weights, m3d, l3d, wg3d,
      ag3d, wv3d, ev3d)


def kernel(memory_matrix, usage_vector, read_weights, write_weight_prev,
           free_gates, lookup_keys, strengths, write_gate, allocation_gate,
           write_vector, erase_vector):
    r3 = lambda x: x.reshape(B, 1, -1)
    mem_t = jnp.swapaxes(memory_matrix, 1, 2)
    rw_t = jnp.swapaxes(read_weights, 1, 2)
    uu, = _k0_call(r3(usage_vector), r3(write_weight_prev), rw_t,
                   r3(free_gates))
    alloc = _alloc_sc(uu.reshape(B, S))
    z, m, l = _k1_call(mem_t, r3(lookup_keys), r3(strengths))
    new_mem_t, read_vecs = _k3_call(
        mem_t, z, r3(alloc), rw_t, m, l, r3(write_gate),
        r3(allocation_gate), write_vector.reshape(B, D, 1),
        erase_vector.reshape(B, D, 1))
    return jnp.swapaxes(new_mem_t, 1, 2), read_vecs
